# Initial kernel scaffold; baseline (speedup 1.0000x reference)
import jax
import jax.numpy as jnp
from jax.experimental import pallas as pl


def kernel(x):
    raise NotImplementedError("write your pallas kernel here")


# speedup vs baseline: 1.2108x; 1.2108x over previous
import numpy as np
import jax
import jax.numpy as jnp
from jax.experimental import pallas as pl
from jax.experimental.pallas import tpu as pltpu

_N = 1000
_F = 26
_W = 512
_TB = 256


def _onehot_body(xp_ref, o_ref):
    c = pl.program_id(1)
    col = c * _W + jax.lax.broadcasted_iota(jnp.int32, (_TB, _W), 1)
    s0 = xp_ref[0, :, 0:1]
    s1 = xp_ref[0, :, 1:2]
    o_ref[...] = ((col == s0) | (col == s1)).astype(jnp.float32)


def kernel(x):
    squeeze = x.ndim == 1
    if squeeze:
        x = x[None, :]
    b, f = x.shape
    assert f == _F
    ncols = f * _N
    nb = -(-b // _TB)
    nc = -(-ncols // _W)

    if b % _TB:
        x = jnp.pad(x, ((0, nb * _TB - b), (0, 0)))

    shifted = x + (_N * jnp.arange(f, dtype=x.dtype))[None, :]

    starts = np.arange(nc) * _W
    f0 = np.minimum(starts // _N, f - 1)
    f1 = np.minimum((starts + _W - 1) // _N, f - 1)
    xpair = jnp.stack([shifted[:, f0], shifted[:, f1]], axis=-1)
    xpair = jnp.transpose(xpair, (1, 0, 2))

    out = pl.pallas_call(
        _onehot_body,
        grid=(nb, nc),
        in_specs=[pl.BlockSpec((1, _TB, 2), lambda bb, cc: (cc, bb, 0))],
        out_specs=pl.BlockSpec((_TB, _W), lambda bb, cc: (bb, cc)),
        out_shape=jax.ShapeDtypeStruct((nb * _TB, ncols), jnp.float32),
        compiler_params=pltpu.CompilerParams(
            dimension_semantics=("arbitrary", "arbitrary"),
        ),
    )(xpair)

    out = out[:b]
    if squeeze:
        pass
    return out


# speedup vs baseline: 1.6898x; 1.3956x over previous
import jax
import jax.numpy as jnp
from jax.experimental import pallas as pl
from jax.experimental.pallas import tpu as pltpu

_N = 1000
_F = 26
_W = 512
_TB = 256


def _onehot_body(x_ref, o_ref):
    c = pl.program_id(1)
    base = c * _W
    f0 = base // _N
    f1 = jnp.minimum((base + _W - 1) // _N, _F - 1)
    xb = x_ref[...]
    lane = jax.lax.broadcasted_iota(jnp.int32, (_TB, _F), 1)
    xv0 = jnp.sum(jnp.where(lane == f0, xb, 0), axis=1, keepdims=True)
    xv1 = jnp.sum(jnp.where(lane == f1, xb, 0), axis=1, keepdims=True)
    col = jax.lax.broadcasted_iota(jnp.int32, (_TB, _W), 1)
    m0 = (col + (base - f0 * _N)) == xv0
    m1 = (col + (base - f1 * _N)) == xv1
    o_ref[...] = (m0 | m1).astype(jnp.float32)


def kernel(x):
    if x.ndim == 1:
        x = x[None, :]
    b, f = x.shape
    assert f == _F
    ncols = f * _N
    nb = -(-b // _TB)
    nc = -(-ncols // _W)

    if b % _TB:
        x = jnp.pad(x, ((0, nb * _TB - b), (0, 0)))

    out = pl.pallas_call(
        _onehot_body,
        grid=(nb, nc),
        in_specs=[pl.BlockSpec((_TB, _F), lambda bb, cc: (bb, 0))],
        out_specs=pl.BlockSpec((_TB, _W), lambda bb, cc: (bb, cc)),
        out_shape=jax.ShapeDtypeStruct((nb * _TB, ncols), jnp.float32),
        compiler_params=pltpu.CompilerParams(
            dimension_semantics=("arbitrary", "arbitrary"),
        ),
    )(x)

    return out[:b]


# speedup vs baseline: 2.0765x; 1.2288x over previous
import jax
import jax.numpy as jnp
from jax.experimental import pallas as pl
from jax.experimental.pallas import tpu as pltpu

_N = 1000
_F = 26
_W = 512
_TB = 512


def _onehot_body(x_ref, o_ref):
    c = pl.program_id(1)
    base = c * _W
    f0 = base // _N
    f1 = jnp.minimum((base + _W - 1) // _N, _F - 1)
    xb = x_ref[...]
    lane = jax.lax.broadcasted_iota(jnp.int32, (_TB, _F), 1)
    xv0 = jnp.sum(jnp.where(lane == f0, xb, 0), axis=1, keepdims=True)
    xv1 = jnp.sum(jnp.where(lane == f1, xb, 0), axis=1, keepdims=True)
    col = jax.lax.broadcasted_iota(jnp.int32, (_TB, _W), 1)
    m0 = (col + (base - f0 * _N)) == xv0
    m1 = (col + (base - f1 * _N)) == xv1
    o_ref[...] = (m0 | m1).astype(jnp.float32)


def kernel(x):
    if x.ndim == 1:
        x = x[None, :]
    b, f = x.shape
    assert f == _F
    ncols = f * _N
    nb = -(-b // _TB)
    nc = -(-ncols // _W)

    if b % _TB:
        x = jnp.pad(x, ((0, nb * _TB - b), (0, 0)))

    out = pl.pallas_call(
        _onehot_body,
        grid=(nb, nc),
        in_specs=[pl.BlockSpec((_TB, _F), lambda bb, cc: (bb, 0))],
        out_specs=pl.BlockSpec((_TB, _W), lambda bb, cc: (bb, cc)),
        out_shape=jax.ShapeDtypeStruct((nb * _TB, ncols), jnp.float32),
        compiler_params=pltpu.CompilerParams(
            dimension_semantics=("arbitrary", "arbitrary"),
        ),
    )(x)

    return out[:b]


# speedup vs baseline: 2.3441x; 1.1289x over previous
import jax
import jax.numpy as jnp
from jax.experimental import pallas as pl
from jax.experimental.pallas import tpu as pltpu

_N = 1000
_F = 26
_W = 512
_TB = 1024


def _onehot_body(x_ref, o_ref):
    c = pl.program_id(1)
    base = c * _W
    f0 = base // _N
    f1 = jnp.minimum((base + _W - 1) // _N, _F - 1)
    xb = x_ref[...]
    lane = jax.lax.broadcasted_iota(jnp.int32, (_TB, _F), 1)
    xv0 = jnp.sum(jnp.where(lane == f0, xb, 0), axis=1, keepdims=True)
    xv1 = jnp.sum(jnp.where(lane == f1, xb, 0), axis=1, keepdims=True)
    col = jax.lax.broadcasted_iota(jnp.int32, (_TB, _W), 1)
    m0 = (col + (base - f0 * _N)) == xv0
    m1 = (col + (base - f1 * _N)) == xv1
    o_ref[...] = (m0 | m1).astype(jnp.float32)


def kernel(x):
    if x.ndim == 1:
        x = x[None, :]
    b, f = x.shape
    assert f == _F
    ncols = f * _N
    nb = -(-b // _TB)
    nc = -(-ncols // _W)

    if b % _TB:
        x = jnp.pad(x, ((0, nb * _TB - b), (0, 0)))

    out = pl.pallas_call(
        _onehot_body,
        grid=(nb, nc),
        in_specs=[pl.BlockSpec((_TB, _F), lambda bb, cc: (bb, 0))],
        out_specs=pl.BlockSpec((_TB, _W), lambda bb, cc: (bb, cc)),
        out_shape=jax.ShapeDtypeStruct((nb * _TB, ncols), jnp.float32),
        compiler_params=pltpu.CompilerParams(
            dimension_semantics=("arbitrary", "arbitrary"),
        ),
    )(x)

    return out[:b]


# speedup vs baseline: 2.5154x; 1.0731x over previous
import jax
import jax.numpy as jnp
from jax.experimental import pallas as pl
from jax.experimental.pallas import tpu as pltpu

_N = 1000
_F = 26
_W = 512
_TB = 2048


def _onehot_body(x_ref, o_ref):
    c = pl.program_id(1)
    base = c * _W
    f0 = base // _N
    f1 = jnp.minimum((base + _W - 1) // _N, _F - 1)
    xb = x_ref[...]
    lane = jax.lax.broadcasted_iota(jnp.int32, (_TB, _F), 1)
    xv0 = jnp.sum(jnp.where(lane == f0, xb, 0), axis=1, keepdims=True)
    xv1 = jnp.sum(jnp.where(lane == f1, xb, 0), axis=1, keepdims=True)
    col = jax.lax.broadcasted_iota(jnp.int32, (_TB, _W), 1)
    m0 = (col + (base - f0 * _N)) == xv0
    m1 = (col + (base - f1 * _N)) == xv1
    o_ref[...] = (m0 | m1).astype(jnp.float32)


def kernel(x):
    if x.ndim == 1:
        x = x[None, :]
    b, f = x.shape
    assert f == _F
    ncols = f * _N
    nb = -(-b // _TB)
    nc = -(-ncols // _W)

    if b % _TB:
        x = jnp.pad(x, ((0, nb * _TB - b), (0, 0)))

    out = pl.pallas_call(
        _onehot_body,
        grid=(nb, nc),
        in_specs=[pl.BlockSpec((_TB, _F), lambda bb, cc: (bb, 0))],
        out_specs=pl.BlockSpec((_TB, _W), lambda bb, cc: (bb, cc)),
        out_shape=jax.ShapeDtypeStruct((nb * _TB, ncols), jnp.float32),
        compiler_params=pltpu.CompilerParams(
            dimension_semantics=("arbitrary", "arbitrary"),
        ),
    )(x)

    return out[:b]


# speedup vs baseline: 2.6026x; 1.0347x over previous
import jax
import jax.numpy as jnp
from jax.experimental import pallas as pl
from jax.experimental.pallas import tpu as pltpu

_N = 1000
_F = 26
_W = 512
_TB = 4096


def _onehot_body(x_ref, o_ref):
    c = pl.program_id(1)
    base = c * _W
    f0 = base // _N
    f1 = jnp.minimum((base + _W - 1) // _N, _F - 1)
    xb = x_ref[...]
    lane = jax.lax.broadcasted_iota(jnp.int32, (_TB, _F), 1)
    xv0 = jnp.sum(jnp.where(lane == f0, xb, 0), axis=1, keepdims=True)
    xv1 = jnp.sum(jnp.where(lane == f1, xb, 0), axis=1, keepdims=True)
    col = jax.lax.broadcasted_iota(jnp.int32, (_TB, _W), 1)
    m0 = (col + (base - f0 * _N)) == xv0
    m1 = (col + (base - f1 * _N)) == xv1
    o_ref[...] = (m0 | m1).astype(jnp.float32)


def kernel(x):
    if x.ndim == 1:
        x = x[None, :]
    b, f = x.shape
    assert f == _F
    ncols = f * _N
    nb = -(-b // _TB)
    nc = -(-ncols // _W)

    if b % _TB:
        x = jnp.pad(x, ((0, nb * _TB - b), (0, 0)))

    out = pl.pallas_call(
        _onehot_body,
        grid=(nb, nc),
        in_specs=[pl.BlockSpec((_TB, _F), lambda bb, cc: (bb, 0))],
        out_specs=pl.BlockSpec((_TB, _W), lambda bb, cc: (bb, cc)),
        out_shape=jax.ShapeDtypeStruct((nb * _TB, ncols), jnp.float32),
        compiler_params=pltpu.CompilerParams(
            dimension_semantics=("arbitrary", "arbitrary"),
        ),
    )(x)

    return out[:b]


# speedup vs baseline: 2.6150x; 1.0048x over previous
import jax
import jax.numpy as jnp
from jax.experimental import pallas as pl
from jax.experimental.pallas import tpu as pltpu

_N = 1000
_F = 26
_W = 512
_TB = 4096


_K = (_W - 2) // _N + 2


def _onehot_body(x_ref, o_ref):
    c = pl.program_id(1)
    base = c * _W
    f0 = base // _N
    xb = x_ref[...]
    lane = jax.lax.broadcasted_iota(jnp.int32, (_TB, _F), 1)
    col = jax.lax.broadcasted_iota(jnp.int32, (_TB, _W), 1)
    m = None
    for k in range(_K):
        fk = jnp.minimum(f0 + k, _F - 1)
        xv = jnp.sum(jnp.where(lane == fk, xb, 0), axis=1, keepdims=True)
        mk = (col + (base - fk * _N)) == xv
        m = mk if m is None else (m | mk)
    o_ref[...] = m.astype(jnp.float32)


def kernel(x):
    if x.ndim == 1:
        x = x[None, :]
    b, f = x.shape
    assert f == _F
    ncols = f * _N
    nb = -(-b // _TB)
    nc = -(-ncols // _W)

    if b % _TB:
        x = jnp.pad(x, ((0, nb * _TB - b), (0, 0)))

    out = pl.pallas_call(
        _onehot_body,
        grid=(nb, nc),
        in_specs=[pl.BlockSpec((_TB, _F), lambda bb, cc: (bb, 0))],
        out_specs=pl.BlockSpec((_TB, _W), lambda bb, cc: (bb, cc)),
        out_shape=jax.ShapeDtypeStruct((nb * _TB, ncols), jnp.float32),
        compiler_params=pltpu.CompilerParams(
            dimension_semantics=("arbitrary", "arbitrary"),
        ),
    )(x)

    return out[:b]
